# Initial kernel scaffold; baseline (speedup 1.0000x reference)
import jax
import jax.numpy as jnp
from jax.experimental import pallas as pl


def kernel(coord, samples):
    raise NotImplementedError("write your pallas kernel here")


# speedup vs baseline: 8.5597x; 8.5597x over previous
import functools

import jax
import jax.numpy as jnp
from jax.experimental import pallas as pl

_RADIUS = 0.2
_NSAMPLE = 32


def _ball_query_kernel(coord_ref, samples_ref, out_ref, *, n, bs):
    cs = coord_ref[0]
    sm = samples_ref[0]

    mm = jax.lax.dot_general(
        sm, cs, (((0,), (0,)), ((), ())),
        preferred_element_type=jnp.float32)
    s2 = jnp.transpose(jnp.sum(sm * sm, axis=0, keepdims=True))
    c2 = jnp.sum(cs * cs, axis=0, keepdims=True)
    d = (-2.0 * mm + s2) + c2

    mask = (d <= (_RADIUS * _RADIUS)).astype(jnp.int32)
    csum = mask
    shift = 1
    while shift < n:
        z = jnp.zeros((bs, shift), jnp.int32)
        csum = csum + jnp.concatenate([z, csum[:, : n - shift]], axis=1)
        shift *= 2

    cols = []
    for k in range(_NSAMPLE):
        cnt = jnp.sum((csum <= k).astype(jnp.int32), axis=1, keepdims=True)
        cols.append(cnt)
    res = jnp.concatenate(cols, axis=1)

    first = res[:, 0:1]
    res = jnp.where(res == n, jnp.broadcast_to(first, res.shape), res)
    out_ref[0] = res


def kernel(coord, samples):
    b, n, _ = coord.shape
    s = samples.shape[1]
    bs = 128

    coord_t = jnp.transpose(coord, (0, 2, 1))
    samples_t = jnp.transpose(samples, (0, 2, 1))

    grid = (b, s // bs)
    out = pl.pallas_call(
        functools.partial(_ball_query_kernel, n=n, bs=bs),
        grid=grid,
        in_specs=[
            pl.BlockSpec((1, 3, n), lambda i, j: (i, 0, 0)),
            pl.BlockSpec((1, 3, bs), lambda i, j: (i, 0, j)),
        ],
        out_specs=pl.BlockSpec((1, bs, _NSAMPLE), lambda i, j: (i, j, 0)),
        out_shape=jax.ShapeDtypeStruct((b, s, _NSAMPLE), jnp.int32),
    )(coord_t, samples_t)
    return out


# speedup vs baseline: 10.7562x; 1.2566x over previous
import functools

import jax
import jax.numpy as jnp
from jax.experimental import pallas as pl
from jax.experimental.pallas import tpu as pltpu

_RADIUS = 0.2
_NSAMPLE = 32
_W = 512
_BS = 128
_KG = 4


def _ball_query_kernel(coord_ref, samples_ref, out_ref, acc_ref, carry_ref,
                       *, n, bs):
    w = _W
    cs = coord_ref[0]
    sm = samples_ref[0]
    s2 = jnp.transpose(jnp.sum(sm * sm, axis=0, keepdims=True))
    c2 = jnp.sum(cs * cs, axis=0, keepdims=True)

    acc_ref[...] = jnp.zeros_like(acc_ref)
    carry_ref[...] = jnp.zeros_like(carry_ref)

    for t in range(n // w):
        cmn = jnp.min(carry_ref[...])

        @pl.when(cmn < float(_NSAMPLE))
        def _(t=t, cmn=cmn):
            csp = cs[:, t * w:(t + 1) * w]
            mm = jax.lax.dot_general(
                sm, csp, (((0,), (0,)), ((), ())),
                preferred_element_type=jnp.float32)
            d = (-2.0 * mm + s2) + c2[:, t * w:(t + 1) * w]
            mk = jnp.where(d <= _RADIUS * _RADIUS, 1.0, 0.0)
            loc = mk
            sft = 1
            while sft < w:
                loc = loc + jnp.concatenate(
                    [jnp.zeros((bs, sft), jnp.float32), loc[:, : w - sft]],
                    axis=1)
                sft *= 2
            csum = loc + carry_ref[...][:, 0:1]
            for k0 in range(0, _NSAMPLE, _KG):
                @pl.when(cmn <= float(k0 + _KG - 1))
                def _(k0=k0, csum=csum):
                    for k in range(k0, k0 + _KG):
                        ind = jnp.where(csum <= float(k), 1.0, 0.0)
                        part = ind[:, 0:128]
                        for q in range(1, w // 128):
                            part = part + ind[:, q * 128:(q + 1) * 128]
                        acc_ref[k] += part
            carry_ref[...] = jnp.broadcast_to(csum[:, w - 1:w],
                                              carry_ref.shape)

    cols = []
    for k in range(_NSAMPLE):
        cols.append(jnp.sum(acc_ref[k], axis=1, keepdims=True))
    res = jnp.concatenate(cols, axis=1).astype(jnp.int32)
    first = res[:, 0:1]
    res = jnp.where(res == n, jnp.broadcast_to(first, res.shape), res)
    out_ref[0] = res


def kernel(coord, samples):
    b, n, _ = coord.shape
    s = samples.shape[1]
    bs = _BS

    r = _RADIUS
    ov = jnp.clip(jnp.minimum(samples + r, 1.0) - jnp.maximum(samples - r, 0.0),
                  0.0, None)
    score = ov[..., 0] * ov[..., 1] * ov[..., 2]
    order = jnp.argsort(-score, axis=1)
    inv = jnp.argsort(order, axis=1)
    samples_s = jnp.take_along_axis(samples, order[..., None], axis=1)

    coord_t = jnp.transpose(coord, (0, 2, 1))
    samples_t = jnp.transpose(samples_s, (0, 2, 1))

    grid = (b, s // bs)
    out = pl.pallas_call(
        functools.partial(_ball_query_kernel, n=n, bs=bs),
        grid=grid,
        in_specs=[
            pl.BlockSpec((1, 3, n), lambda i, j: (i, 0, 0)),
            pl.BlockSpec((1, 3, bs), lambda i, j: (i, 0, j)),
        ],
        out_specs=pl.BlockSpec((1, bs, _NSAMPLE), lambda i, j: (i, j, 0)),
        out_shape=jax.ShapeDtypeStruct((b, s, _NSAMPLE), jnp.int32),
        scratch_shapes=[
            pltpu.VMEM((_NSAMPLE, bs, 128), jnp.float32),
            pltpu.VMEM((bs, 128), jnp.float32),
        ],
    )(coord_t, samples_t)
    return jnp.take_along_axis(out, inv[..., None], axis=1)
